# baseline (speedup 1.0000x reference)
import functools

import jax
import jax.numpy as jnp
from jax import lax
from jax.experimental import pallas as pl
from jax.experimental.pallas import tpu as pltpu
from jax.experimental.pallas import tpu_sc as plsc

L = 16
NC = 2
NS = 16
CHUNK = 128
HALF = 128

_MESH = plsc.VectorSubcoreMesh(core_axis_name="c", subcore_axis_name="s")
_SC_PARAMS = pltpu.CompilerParams(needs_layout_passes=False)


def _sc_deg(col_flat, w_flat, npad):
    epad = col_flat.shape[0]
    ept = epad // (NC * NS)

    @functools.partial(
        pl.kernel,
        out_type=jax.ShapeDtypeStruct((NC * NS * npad,), jnp.float32),
        mesh=_MESH,
        compiler_params=_SC_PARAMS,
        scratch_types=[
            pltpu.VMEM((npad,), jnp.float32),
            pltpu.VMEM((ept,), jnp.int32),
            pltpu.VMEM((ept,), jnp.float32),
        ],
    )
    def deg_kernel(col_hbm, w_hbm, degp_hbm, deg_v, coli, wv):
        cid = lax.axis_index("c")
        sid = lax.axis_index("s")
        wid = cid * NS + sid
        zero = jnp.zeros((L,), jnp.float32)

        pltpu.sync_copy(col_hbm.at[pl.ds(wid * ept, ept)], coli)
        pltpu.sync_copy(w_hbm.at[pl.ds(wid * ept, ept)], wv)

        @pl.loop(0, npad // L)
        def _(g):
            deg_v[pl.ds(g * L, L)] = zero

        @pl.loop(0, ept // L)
        def _(g):
            sl = pl.ds(g * L, L)
            plsc.addupdate_scatter(deg_v, [coli[sl]], wv[sl])

        pltpu.sync_copy(deg_v, degp_hbm.at[pl.ds(wid * npad, npad)])

    return deg_kernel(col_flat, w_flat)


MCHUNK = 64
IBLK = 16


def _sc_msg(y_flat, row_flat, col3d, w_flat, npad):
    nchunks = col3d.shape[0]
    cpt = nchunks // NS
    nblk = cpt // IBLK
    eblk = IBLK * MCHUNK
    rpt = npad // NS

    @functools.partial(
        pl.kernel,
        out_type=jax.ShapeDtypeStruct((NC * npad, HALF), jnp.float32),
        mesh=_MESH,
        compiler_params=_SC_PARAMS,
        scratch_types=[
            pltpu.VMEM_SHARED((npad, HALF), jnp.float32),
            pltpu.VMEM((2, eblk), jnp.int32),
            pltpu.VMEM((2, IBLK, 1, MCHUNK), jnp.int32),
            pltpu.VMEM((2, eblk), jnp.float32),
            pltpu.VMEM((3, MCHUNK, HALF), jnp.float32),
            pltpu.SemaphoreType.DMA((3,)),
            pltpu.SemaphoreType.DMA((3,)),
        ],
    )
    def msg_kernel(y_hbm, row_hbm, col_hbm, w_hbm, out_hbm,
                   acc_sp, rowi, coli, wv, bufs, gsem, ssem):
        cid = lax.axis_index("c")
        sid = lax.axis_index("s")
        row0 = sid * rpt
        slab0 = cid * npad
        base = sid * cpt
        offv = jnp.full((L,), slab0, dtype=jnp.int32)

        def load_block(b):
            sb = lax.rem(b, 2)
            pltpu.sync_copy(row_hbm.at[pl.ds((base + b * IBLK) * MCHUNK, eblk)],
                            rowi.at[sb])
            pltpu.sync_copy(col_hbm.at[pl.ds(base + b * IBLK, IBLK)],
                            coli.at[sb])
            pltpu.sync_copy(w_hbm.at[pl.ds((base + b * IBLK) * MCHUNK, eblk)],
                            wv.at[sb])

            @pl.loop(0, eblk // L)
            def _(g):
                sl = pl.ds(g * L, L)
                rowi[sb, sl] = rowi[sb, sl] + offv

        pltpu.sync_copy(
            y_hbm.at[pl.ds(slab0 + row0, rpt)],
            acc_sp.at[pl.ds(row0, rpt)],
        )
        plsc.subcore_barrier()

        def start_gather(k, slot):
            sb = lax.rem(lax.div(k, IBLK), 2)
            o = lax.rem(k, IBLK) * MCHUNK
            pltpu.async_copy(
                y_hbm.at[rowi.at[sb, pl.ds(o, MCHUNK)]], bufs.at[slot],
                gsem.at[slot])

        def wait_gather(slot):
            pltpu.make_async_copy(
                y_hbm.at[rowi.at[0, pl.ds(0, MCHUNK)]], bufs.at[slot],
                gsem.at[slot]).wait()

        def start_scatter(k, slot):
            sb = lax.rem(lax.div(k, IBLK), 2)
            j = lax.rem(k, IBLK)
            pltpu.async_copy(
                bufs.at[slot], acc_sp.at[coli.at[sb, j, 0]], ssem.at[slot],
                add=True)

        def wait_scatter(slot):
            pltpu.make_async_copy(
                bufs.at[slot], acc_sp.at[coli.at[0, 0, 0]],
                ssem.at[slot]).wait()

        def scale(k, slot):
            sb = lax.rem(lax.div(k, IBLK), 2)
            o = lax.rem(k, IBLK) * MCHUNK
            sbv = jnp.full((L,), sb, dtype=jnp.int32)

            @pl.loop(0, MCHUNK, step=8)
            def _(e):
                for u in range(8):
                    idx = jnp.full((L,), o + e + u, dtype=jnp.int32)
                    wsp = plsc.load_gather(wv, [sbv, idx])
                    for f in range(HALF // L):
                        sl = pl.ds(f * L, L)
                        bufs[slot, e + u, sl] = bufs[slot, e + u, sl] * wsp

        load_block(0)
        start_gather(0, 0)
        start_gather(1, 1)

        @pl.loop(0, cpt)
        def _(k):
            @pl.when(jnp.logical_and(lax.rem(k, IBLK) == IBLK - 2,
                                     lax.div(k, IBLK) + 1 < nblk))
            def _():
                load_block(lax.div(k, IBLK) + 1)

            slot = lax.rem(k, 3)
            wait_gather(slot)
            scale(k, slot)
            start_scatter(k, slot)
            slot_r = lax.rem(k + 2, 3)

            @pl.when(k >= 1)
            def _():
                wait_scatter(slot_r)

            @pl.when(k + 2 < cpt)
            def _():
                start_gather(k + 2, slot_r)

        wait_scatter(lax.rem(cpt - 1, 3))

        plsc.subcore_barrier()
        pltpu.sync_copy(
            acc_sp.at[pl.ds(row0, rpt)],
            out_hbm.at[pl.ds(slab0 + row0, rpt)],
        )

    return msg_kernel(y_flat, row_flat, col3d, w_flat)


def _dis_from_degp(degp_blk):
    deg = jnp.sum(degp_blk, axis=0) + 1.0
    return jnp.where(deg > 0, lax.rsqrt(jnp.maximum(deg, 1e-12)), 0.0)


def _tc_y(x_pad, W, degp2):
    npad, d_in = x_pad.shape
    d_out = W.shape[0]
    br = npad // 8

    def body(x_ref, w_ref, degp_ref, y_ref):
        xl = lax.dot_general(
            x_ref[...], w_ref[...], (((1,), (1,)), ((), ())),
            preferred_element_type=jnp.float32,
            precision=lax.Precision.HIGHEST,
        )
        y = xl * _dis_from_degp(degp_ref[...])[:, None]
        y_ref[...] = jnp.stack([y[:, :HALF], y[:, HALF:]], axis=0)

    return pl.pallas_call(
        body,
        grid=(npad // br,),
        in_specs=[
            pl.BlockSpec((br, d_in), lambda i: (i, 0)),
            pl.BlockSpec((d_out, d_in), lambda i: (0, 0)),
            pl.BlockSpec((NC * NS, br), lambda i: (0, i)),
        ],
        out_specs=pl.BlockSpec((NC, br, HALF), lambda i: (0, i, 0)),
        out_shape=jax.ShapeDtypeStruct((NC, npad, HALF), jnp.float32),
    )(x_pad, W, degp2)


def _tc_final(acc3, degp2, b2d):
    npad = acc3.shape[1]
    d_out = NC * HALF
    br = npad // 10

    def body(acc_ref, degp_ref, b_ref, o_ref):
        dis = _dis_from_degp(degp_ref[...])
        m = jnp.concatenate([acc_ref[0], acc_ref[1]], axis=1)
        o_ref[...] = m * dis[:, None] + b_ref[...]

    return pl.pallas_call(
        body,
        grid=(npad // br,),
        in_specs=[
            pl.BlockSpec((NC, br, HALF), lambda i: (0, i, 0)),
            pl.BlockSpec((NC * NS, br), lambda i: (0, i)),
            pl.BlockSpec((1, d_out), lambda i: (0, 0)),
        ],
        out_specs=pl.BlockSpec((br, d_out), lambda i: (i, 0)),
        out_shape=jax.ShapeDtypeStruct((npad, d_out), jnp.float32),
    )(acc3, degp2, b2d)


def kernel(x, edge_index, edge_weight, W, b):
    n = x.shape[0]
    e = edge_weight.shape[0]
    npad = -(-n // (NS * L)) * (NS * L)
    row = edge_index[0].astype(jnp.int32)
    col = edge_index[1].astype(jnp.int32)
    w = edge_weight.astype(jnp.float32)

    egrain = NS * MCHUNK * IBLK
    epad = -(-e // egrain) * egrain
    pad = epad - e
    pad_idx = (jnp.arange(pad, dtype=jnp.int32) * 37) % n
    row_flat = jnp.concatenate([row, pad_idx])
    col_flat = jnp.concatenate([col, pad_idx])
    col3d = col_flat.reshape(epad // MCHUNK, 1, MCHUNK)
    w_flat = jnp.concatenate([w, jnp.zeros((pad,), jnp.float32)])
    x_pad = jnp.concatenate(
        [x, jnp.zeros((npad - n, x.shape[1]), x.dtype)], axis=0)

    degp = _sc_deg(col_flat, w_flat, npad)
    degp2 = degp.reshape(NC * NS, npad)
    y = _tc_y(x_pad, W, degp2)
    acc = _sc_msg(y.reshape(NC * npad, HALF), row_flat, col3d, w_flat, npad)
    out = _tc_final(acc.reshape(NC, npad, HALF), degp2,
                    b.reshape(1, NC * HALF))
    return out[:n]


# speedup vs baseline: 2.0733x; 2.0733x over previous
import functools

import jax
import jax.numpy as jnp
from jax import lax
from jax.experimental import pallas as pl
from jax.experimental.pallas import tpu as pltpu
from jax.experimental.pallas import tpu_sc as plsc

L = 16
NC = 2
NS = 16
CHUNK = 128
HALF = 128

_MESH = plsc.VectorSubcoreMesh(core_axis_name="c", subcore_axis_name="s")
_SC_PARAMS = pltpu.CompilerParams(needs_layout_passes=False)


def _sc_deg(col_flat, w_flat, npad):
    epad = col_flat.shape[0]
    ept = epad // (NC * NS)

    @functools.partial(
        pl.kernel,
        out_type=jax.ShapeDtypeStruct((NC * NS * npad,), jnp.float32),
        mesh=_MESH,
        compiler_params=_SC_PARAMS,
        scratch_types=[
            pltpu.VMEM((npad,), jnp.float32),
            pltpu.VMEM((ept,), jnp.int32),
            pltpu.VMEM((ept,), jnp.float32),
        ],
    )
    def deg_kernel(col_hbm, w_hbm, degp_hbm, deg_v, coli, wv):
        cid = lax.axis_index("c")
        sid = lax.axis_index("s")
        wid = cid * NS + sid
        zero = jnp.zeros((L,), jnp.float32)

        pltpu.sync_copy(col_hbm.at[pl.ds(wid * ept, ept)], coli)
        pltpu.sync_copy(w_hbm.at[pl.ds(wid * ept, ept)], wv)

        @pl.loop(0, npad // L)
        def _(g):
            deg_v[pl.ds(g * L, L)] = zero

        @pl.loop(0, ept // L)
        def _(g):
            sl = pl.ds(g * L, L)
            plsc.addupdate_scatter(deg_v, [coli[sl]], wv[sl])

        pltpu.sync_copy(deg_v, degp_hbm.at[pl.ds(wid * npad, npad)])

    return deg_kernel(col_flat, w_flat)


MCHUNK = 64
IBLK = 16


def _sc_msg(y_flat, row_flat, col3d, w_flat, npad):
    nchunks = col3d.shape[0]
    cpt = nchunks // NS
    nblk = cpt // IBLK
    eblk = IBLK * MCHUNK
    rpt = npad // NS

    @functools.partial(
        pl.kernel,
        out_type=jax.ShapeDtypeStruct((NC * npad, HALF), jnp.float32),
        mesh=_MESH,
        compiler_params=_SC_PARAMS,
        scratch_types=[
            pltpu.VMEM_SHARED((npad, HALF), jnp.float32),
            pltpu.VMEM((2, eblk), jnp.int32),
            pltpu.VMEM((2, IBLK, 1, MCHUNK), jnp.int32),
            pltpu.VMEM((2, eblk), jnp.float32),
            pltpu.VMEM((3, MCHUNK, HALF), jnp.float32),
            pltpu.SemaphoreType.DMA((3,)),
            pltpu.SemaphoreType.DMA((3,)),
        ],
    )
    def msg_kernel(y_hbm, row_hbm, col_hbm, w_hbm, out_hbm,
                   acc_sp, rowi, coli, wv, bufs, gsem, ssem):
        cid = lax.axis_index("c")
        sid = lax.axis_index("s")
        row0 = sid * rpt
        slab0 = cid * npad
        base = sid * cpt
        offv = jnp.full((L,), slab0, dtype=jnp.int32)

        def load_block(b):
            sb = lax.rem(b, 2)
            pltpu.sync_copy(row_hbm.at[pl.ds((base + b * IBLK) * MCHUNK, eblk)],
                            rowi.at[sb])
            pltpu.sync_copy(col_hbm.at[pl.ds(base + b * IBLK, IBLK)],
                            coli.at[sb])
            pltpu.sync_copy(w_hbm.at[pl.ds((base + b * IBLK) * MCHUNK, eblk)],
                            wv.at[sb])

            @pl.loop(0, eblk // L)
            def _(g):
                sl = pl.ds(g * L, L)
                rowi[sb, sl] = rowi[sb, sl] + offv

        pltpu.sync_copy(
            y_hbm.at[pl.ds(slab0 + row0, rpt)],
            acc_sp.at[pl.ds(row0, rpt)],
        )
        plsc.subcore_barrier()

        def start_gather(k, slot):
            sb = lax.rem(lax.div(k, IBLK), 2)
            o = lax.rem(k, IBLK) * MCHUNK
            pltpu.async_copy(
                y_hbm.at[rowi.at[sb, pl.ds(o, MCHUNK)]], bufs.at[slot],
                gsem.at[slot])

        def wait_gather(slot):
            pltpu.make_async_copy(
                y_hbm.at[rowi.at[0, pl.ds(0, MCHUNK)]], bufs.at[slot],
                gsem.at[slot]).wait()

        def start_scatter(k, slot):
            sb = lax.rem(lax.div(k, IBLK), 2)
            j = lax.rem(k, IBLK)
            pltpu.async_copy(
                bufs.at[slot], acc_sp.at[coli.at[sb, j, 0]], ssem.at[slot],
                add=True)

        def wait_scatter(slot):
            pltpu.make_async_copy(
                bufs.at[slot], acc_sp.at[coli.at[0, 0, 0]],
                ssem.at[slot]).wait()

        def scale(k, slot):
            sb = lax.rem(lax.div(k, IBLK), 2)
            o = lax.rem(k, IBLK) * MCHUNK
            sbv = jnp.full((L,), sb, dtype=jnp.int32)

            @pl.loop(0, MCHUNK, step=4)
            def _(e):
                for u in range(4):
                    idx = jnp.full((L,), o + e + u, dtype=jnp.int32)
                    wsp = plsc.load_gather(wv, [sbv, idx])
                    for f in range(HALF // L):
                        sl = pl.ds(f * L, L)
                        bufs[slot, e + u, sl] = bufs[slot, e + u, sl] * wsp

        load_block(0)
        start_gather(0, 0)
        start_gather(1, 1)

        @pl.loop(0, cpt)
        def _(k):
            @pl.when(jnp.logical_and(lax.rem(k, IBLK) == IBLK - 2,
                                     lax.div(k, IBLK) + 1 < nblk))
            def _():
                load_block(lax.div(k, IBLK) + 1)

            slot = lax.rem(k, 3)
            wait_gather(slot)
            scale(k, slot)
            start_scatter(k, slot)
            slot_r = lax.rem(k + 2, 3)

            @pl.when(k >= 1)
            def _():
                wait_scatter(slot_r)

            @pl.when(k + 2 < cpt)
            def _():
                start_gather(k + 2, slot_r)

        wait_scatter(lax.rem(cpt - 1, 3))

        plsc.subcore_barrier()
        pltpu.sync_copy(
            acc_sp.at[pl.ds(row0, rpt)],
            out_hbm.at[pl.ds(slab0 + row0, rpt)],
        )

    return msg_kernel(y_flat, row_flat, col3d, w_flat)


def _dis_from_degp(degp_blk):
    deg = jnp.sum(degp_blk, axis=0) + 1.0
    return jnp.where(deg > 0, lax.rsqrt(jnp.maximum(deg, 1e-12)), 0.0)


def _tc_y(x_pad, W, degp2):
    npad, d_in = x_pad.shape
    d_out = W.shape[0]
    br = npad // 8

    def body(x_ref, w_ref, degp_ref, y_ref):
        xl = lax.dot_general(
            x_ref[...], w_ref[...], (((1,), (1,)), ((), ())),
            preferred_element_type=jnp.float32,
            precision=lax.Precision.HIGHEST,
        )
        y = xl * _dis_from_degp(degp_ref[...])[:, None]
        y_ref[...] = jnp.stack([y[:, :HALF], y[:, HALF:]], axis=0)

    return pl.pallas_call(
        body,
        grid=(npad // br,),
        in_specs=[
            pl.BlockSpec((br, d_in), lambda i: (i, 0)),
            pl.BlockSpec((d_out, d_in), lambda i: (0, 0)),
            pl.BlockSpec((NC * NS, br), lambda i: (0, i)),
        ],
        out_specs=pl.BlockSpec((NC, br, HALF), lambda i: (0, i, 0)),
        out_shape=jax.ShapeDtypeStruct((NC, npad, HALF), jnp.float32),
    )(x_pad, W, degp2)


def _tc_final(acc3, degp2, b2d):
    npad = acc3.shape[1]
    d_out = NC * HALF
    br = npad // 10

    def body(acc_ref, degp_ref, b_ref, o_ref):
        dis = _dis_from_degp(degp_ref[...])
        m = jnp.concatenate([acc_ref[0], acc_ref[1]], axis=1)
        o_ref[...] = m * dis[:, None] + b_ref[...]

    return pl.pallas_call(
        body,
        grid=(npad // br,),
        in_specs=[
            pl.BlockSpec((NC, br, HALF), lambda i: (0, i, 0)),
            pl.BlockSpec((NC * NS, br), lambda i: (0, i)),
            pl.BlockSpec((1, d_out), lambda i: (0, 0)),
        ],
        out_specs=pl.BlockSpec((br, d_out), lambda i: (i, 0)),
        out_shape=jax.ShapeDtypeStruct((npad, d_out), jnp.float32),
    )(acc3, degp2, b2d)


def kernel(x, edge_index, edge_weight, W, b):
    n = x.shape[0]
    e = edge_weight.shape[0]
    npad = -(-n // (NS * L)) * (NS * L)
    row = edge_index[0].astype(jnp.int32)
    col = edge_index[1].astype(jnp.int32)
    w = edge_weight.astype(jnp.float32)

    egrain = NS * MCHUNK * IBLK
    epad = -(-e // egrain) * egrain
    pad = epad - e
    pad_idx = (jnp.arange(pad, dtype=jnp.int32) * 37) % n
    row_flat = jnp.concatenate([row, pad_idx])
    col_flat = jnp.concatenate([col, pad_idx])
    col3d = col_flat.reshape(epad // MCHUNK, 1, MCHUNK)
    w_flat = jnp.concatenate([w, jnp.zeros((pad,), jnp.float32)])
    x_pad = jnp.concatenate(
        [x, jnp.zeros((npad - n, x.shape[1]), x.dtype)], axis=0)

    degp = _sc_deg(col_flat, w_flat, npad)
    degp2 = degp.reshape(NC * NS, npad)
    y = _tc_y(x_pad, W, degp2)
    acc = _sc_msg(y.reshape(NC * npad, HALF), row_flat, col3d, w_flat, npad)
    out = _tc_final(acc.reshape(NC, npad, HALF), degp2,
                    b.reshape(1, NC * HALF))
    return out[:n]


# speedup vs baseline: 2.1392x; 1.0318x over previous
import functools

import jax
import jax.numpy as jnp
from jax import lax
from jax.experimental import pallas as pl
from jax.experimental.pallas import tpu as pltpu
from jax.experimental.pallas import tpu_sc as plsc

L = 16
NC = 2
NS = 16
CHUNK = 128
HALF = 128

_MESH = plsc.VectorSubcoreMesh(core_axis_name="c", subcore_axis_name="s")
_SC_PARAMS = pltpu.CompilerParams(needs_layout_passes=False)


def _sc_deg(col_flat, w_flat, npad):
    epad = col_flat.shape[0]
    ept = epad // (NC * NS)

    @functools.partial(
        pl.kernel,
        out_type=jax.ShapeDtypeStruct((NC * NS * npad,), jnp.float32),
        mesh=_MESH,
        compiler_params=_SC_PARAMS,
        scratch_types=[
            pltpu.VMEM((npad,), jnp.float32),
            pltpu.VMEM((ept,), jnp.int32),
            pltpu.VMEM((ept,), jnp.float32),
        ],
    )
    def deg_kernel(col_hbm, w_hbm, degp_hbm, deg_v, coli, wv):
        cid = lax.axis_index("c")
        sid = lax.axis_index("s")
        wid = cid * NS + sid
        zero = jnp.zeros((L,), jnp.float32)

        pltpu.sync_copy(col_hbm.at[pl.ds(wid * ept, ept)], coli)
        pltpu.sync_copy(w_hbm.at[pl.ds(wid * ept, ept)], wv)

        @pl.loop(0, npad // L)
        def _(g):
            deg_v[pl.ds(g * L, L)] = zero

        @pl.loop(0, ept // L)
        def _(g):
            sl = pl.ds(g * L, L)
            plsc.addupdate_scatter(deg_v, [coli[sl]], wv[sl])

        pltpu.sync_copy(deg_v, degp_hbm.at[pl.ds(wid * npad, npad)])

    return deg_kernel(col_flat, w_flat)


MCHUNK = 64
IBLK = 40


def _sc_msg(y_flat, row_flat, col3d, w_flat, npad):
    nchunks = col3d.shape[0]
    cpt = nchunks // NS
    nblk = cpt // IBLK
    eblk = IBLK * MCHUNK
    rpt = npad // NS

    @functools.partial(
        pl.kernel,
        out_type=jax.ShapeDtypeStruct((NC * npad, HALF), jnp.float32),
        mesh=_MESH,
        compiler_params=_SC_PARAMS,
        scratch_types=[
            pltpu.VMEM_SHARED((npad, HALF), jnp.float32),
            pltpu.VMEM((2, eblk), jnp.int32),
            pltpu.VMEM((2, IBLK, 1, MCHUNK), jnp.int32),
            pltpu.VMEM((2, eblk), jnp.float32),
            pltpu.VMEM((3, MCHUNK, HALF), jnp.float32),
            pltpu.SemaphoreType.DMA((3,)),
            pltpu.SemaphoreType.DMA((3,)),
        ],
    )
    def msg_kernel(y_hbm, row_hbm, col_hbm, w_hbm, out_hbm,
                   acc_sp, rowi, coli, wv, bufs, gsem, ssem):
        cid = lax.axis_index("c")
        sid = lax.axis_index("s")
        row0 = sid * rpt
        slab0 = cid * npad
        base = sid * cpt
        offv = jnp.full((L,), slab0, dtype=jnp.int32)

        def load_block(b):
            sb = lax.rem(b, 2)
            pltpu.sync_copy(row_hbm.at[pl.ds((base + b * IBLK) * MCHUNK, eblk)],
                            rowi.at[sb])
            pltpu.sync_copy(col_hbm.at[pl.ds(base + b * IBLK, IBLK)],
                            coli.at[sb])
            pltpu.sync_copy(w_hbm.at[pl.ds((base + b * IBLK) * MCHUNK, eblk)],
                            wv.at[sb])

            @pl.loop(0, eblk // L)
            def _(g):
                sl = pl.ds(g * L, L)
                rowi[sb, sl] = rowi[sb, sl] + offv

        pltpu.sync_copy(
            y_hbm.at[pl.ds(slab0 + row0, rpt)],
            acc_sp.at[pl.ds(row0, rpt)],
        )
        plsc.subcore_barrier()

        def start_gather(k, slot):
            sb = lax.rem(lax.div(k, IBLK), 2)
            o = lax.rem(k, IBLK) * MCHUNK
            pltpu.async_copy(
                y_hbm.at[rowi.at[sb, pl.ds(o, MCHUNK)]], bufs.at[slot],
                gsem.at[slot])

        def wait_gather(slot):
            pltpu.make_async_copy(
                y_hbm.at[rowi.at[0, pl.ds(0, MCHUNK)]], bufs.at[slot],
                gsem.at[slot]).wait()

        def start_scatter(k, slot):
            sb = lax.rem(lax.div(k, IBLK), 2)
            j = lax.rem(k, IBLK)
            pltpu.async_copy(
                bufs.at[slot], acc_sp.at[coli.at[sb, j, 0]], ssem.at[slot],
                add=True)

        def wait_scatter(slot):
            pltpu.make_async_copy(
                bufs.at[slot], acc_sp.at[coli.at[0, 0, 0]],
                ssem.at[slot]).wait()

        def scale(k, slot):
            sb = lax.rem(lax.div(k, IBLK), 2)
            o = lax.rem(k, IBLK) * MCHUNK
            sbv = jnp.full((L,), sb, dtype=jnp.int32)

            @pl.loop(0, MCHUNK, step=4)
            def _(e):
                for u in range(4):
                    idx = jnp.full((L,), o + e + u, dtype=jnp.int32)
                    wsp = plsc.load_gather(wv, [sbv, idx])
                    for f in range(HALF // L):
                        sl = pl.ds(f * L, L)
                        bufs[slot, e + u, sl] = bufs[slot, e + u, sl] * wsp

        load_block(0)
        start_gather(0, 0)
        start_gather(1, 1)

        @pl.loop(0, cpt)
        def _(k):
            @pl.when(jnp.logical_and(lax.rem(k, IBLK) == IBLK - 2,
                                     lax.div(k, IBLK) + 1 < nblk))
            def _():
                load_block(lax.div(k, IBLK) + 1)

            slot = lax.rem(k, 3)
            wait_gather(slot)
            scale(k, slot)
            start_scatter(k, slot)
            slot_r = lax.rem(k + 2, 3)

            @pl.when(k >= 1)
            def _():
                wait_scatter(slot_r)

            @pl.when(k + 2 < cpt)
            def _():
                start_gather(k + 2, slot_r)

        wait_scatter(lax.rem(cpt - 1, 3))

        plsc.subcore_barrier()
        pltpu.sync_copy(
            acc_sp.at[pl.ds(row0, rpt)],
            out_hbm.at[pl.ds(slab0 + row0, rpt)],
        )

    return msg_kernel(y_flat, row_flat, col3d, w_flat)


def _dis_from_degp(degp_blk):
    deg = jnp.sum(degp_blk, axis=0) + 1.0
    return jnp.where(deg > 0, lax.rsqrt(jnp.maximum(deg, 1e-12)), 0.0)


def _tc_y(x_pad, W, degp2):
    npad, d_in = x_pad.shape
    d_out = W.shape[0]
    br = npad // 8

    def body(x_ref, w_ref, degp_ref, y_ref):
        xl = lax.dot_general(
            x_ref[...], w_ref[...], (((1,), (1,)), ((), ())),
            preferred_element_type=jnp.float32,
            precision=lax.Precision.HIGHEST,
        )
        y = xl * _dis_from_degp(degp_ref[...])[:, None]
        y_ref[...] = jnp.stack([y[:, :HALF], y[:, HALF:]], axis=0)

    return pl.pallas_call(
        body,
        grid=(npad // br,),
        in_specs=[
            pl.BlockSpec((br, d_in), lambda i: (i, 0)),
            pl.BlockSpec((d_out, d_in), lambda i: (0, 0)),
            pl.BlockSpec((NC * NS, br), lambda i: (0, i)),
        ],
        out_specs=pl.BlockSpec((NC, br, HALF), lambda i: (0, i, 0)),
        out_shape=jax.ShapeDtypeStruct((NC, npad, HALF), jnp.float32),
    )(x_pad, W, degp2)


def _tc_final(acc3, degp2, b2d):
    npad = acc3.shape[1]
    d_out = NC * HALF
    br = npad // 10

    def body(acc_ref, degp_ref, b_ref, o_ref):
        dis = _dis_from_degp(degp_ref[...])
        m = jnp.concatenate([acc_ref[0], acc_ref[1]], axis=1)
        o_ref[...] = m * dis[:, None] + b_ref[...]

    return pl.pallas_call(
        body,
        grid=(npad // br,),
        in_specs=[
            pl.BlockSpec((NC, br, HALF), lambda i: (0, i, 0)),
            pl.BlockSpec((NC * NS, br), lambda i: (0, i)),
            pl.BlockSpec((1, d_out), lambda i: (0, 0)),
        ],
        out_specs=pl.BlockSpec((br, d_out), lambda i: (i, 0)),
        out_shape=jax.ShapeDtypeStruct((npad, d_out), jnp.float32),
    )(acc3, degp2, b2d)


def kernel(x, edge_index, edge_weight, W, b):
    n = x.shape[0]
    e = edge_weight.shape[0]
    npad = -(-n // (NS * L)) * (NS * L)
    row = edge_index[0].astype(jnp.int32)
    col = edge_index[1].astype(jnp.int32)
    w = edge_weight.astype(jnp.float32)

    egrain = NS * MCHUNK * IBLK
    epad = -(-e // egrain) * egrain
    pad = epad - e
    pad_idx = (jnp.arange(pad, dtype=jnp.int32) * 37) % n
    row_flat = jnp.concatenate([row, pad_idx])
    col_flat = jnp.concatenate([col, pad_idx])
    col3d = col_flat.reshape(epad // MCHUNK, 1, MCHUNK)
    w_flat = jnp.concatenate([w, jnp.zeros((pad,), jnp.float32)])
    x_pad = jnp.concatenate(
        [x, jnp.zeros((npad - n, x.shape[1]), x.dtype)], axis=0)

    degp = _sc_deg(col_flat, w_flat, npad)
    degp2 = degp.reshape(NC * NS, npad)
    y = _tc_y(x_pad, W, degp2)
    acc = _sc_msg(y.reshape(NC * npad, HALF), row_flat, col3d, w_flat, npad)
    out = _tc_final(acc.reshape(NC, npad, HALF), degp2,
                    b.reshape(1, NC * HALF))
    return out[:n]


# speedup vs baseline: 2.1692x; 1.0140x over previous
import functools

import jax
import jax.numpy as jnp
from jax import lax
from jax.experimental import pallas as pl
from jax.experimental.pallas import tpu as pltpu
from jax.experimental.pallas import tpu_sc as plsc

L = 16
NC = 2
NS = 16
CHUNK = 128
HALF = 128

_MESH = plsc.VectorSubcoreMesh(core_axis_name="c", subcore_axis_name="s")
_SC_PARAMS = pltpu.CompilerParams(needs_layout_passes=False)


def _sc_deg(col_flat, w_flat, npad):
    epad = col_flat.shape[0]
    ept = epad // (NC * NS)

    @functools.partial(
        pl.kernel,
        out_type=jax.ShapeDtypeStruct((NC * NS * npad,), jnp.float32),
        mesh=_MESH,
        compiler_params=_SC_PARAMS,
        scratch_types=[
            pltpu.VMEM((npad,), jnp.float32),
            pltpu.VMEM((ept,), jnp.int32),
            pltpu.VMEM((ept,), jnp.float32),
        ],
    )
    def deg_kernel(col_hbm, w_hbm, degp_hbm, deg_v, coli, wv):
        cid = lax.axis_index("c")
        sid = lax.axis_index("s")
        wid = cid * NS + sid
        zero = jnp.zeros((L,), jnp.float32)

        pltpu.sync_copy(col_hbm.at[pl.ds(wid * ept, ept)], coli)
        pltpu.sync_copy(w_hbm.at[pl.ds(wid * ept, ept)], wv)

        @pl.loop(0, npad // L)
        def _(g):
            deg_v[pl.ds(g * L, L)] = zero

        @pl.loop(0, ept // L)
        def _(g):
            sl = pl.ds(g * L, L)
            plsc.addupdate_scatter(deg_v, [coli[sl]], wv[sl])

        pltpu.sync_copy(deg_v, degp_hbm.at[pl.ds(wid * npad, npad)])

    return deg_kernel(col_flat, w_flat)


MCHUNK = 64
IBLK = 40


def _sc_msg(y_flat, row_flat, col3d, w_flat, npad):
    nchunks = col3d.shape[0]
    cpt = nchunks // NS
    nblk = cpt // IBLK
    eblk = IBLK * MCHUNK
    rpt = npad // NS

    @functools.partial(
        pl.kernel,
        out_type=jax.ShapeDtypeStruct((NC * npad, HALF), jnp.float32),
        mesh=_MESH,
        compiler_params=_SC_PARAMS,
        scratch_types=[
            pltpu.VMEM_SHARED((npad, HALF), jnp.float32),
            pltpu.VMEM((2, eblk), jnp.int32),
            pltpu.VMEM((2, IBLK, 1, MCHUNK), jnp.int32),
            pltpu.VMEM((2, eblk), jnp.float32),
            pltpu.VMEM((3, MCHUNK, HALF), jnp.float32),
            pltpu.SemaphoreType.DMA((3,)),
            pltpu.SemaphoreType.DMA((3,)),
        ],
    )
    def msg_kernel(y_hbm, row_hbm, col_hbm, w_hbm, out_hbm,
                   acc_sp, rowi, coli, wv, bufs, gsem, ssem):
        cid = lax.axis_index("c")
        sid = lax.axis_index("s")
        row0 = sid * rpt
        slab0 = cid * npad
        base = sid * cpt
        offv = jnp.full((L,), slab0, dtype=jnp.int32)

        def load_block(b):
            sb = lax.rem(b, 2)
            pltpu.sync_copy(row_hbm.at[pl.ds((base + b * IBLK) * MCHUNK, eblk)],
                            rowi.at[sb])
            pltpu.sync_copy(col_hbm.at[pl.ds(base + b * IBLK, IBLK)],
                            coli.at[sb])
            pltpu.sync_copy(w_hbm.at[pl.ds((base + b * IBLK) * MCHUNK, eblk)],
                            wv.at[sb])

            @pl.loop(0, eblk // L)
            def _(g):
                sl = pl.ds(g * L, L)
                rowi[sb, sl] = rowi[sb, sl] + offv

        pltpu.sync_copy(
            y_hbm.at[pl.ds(slab0 + row0, rpt)],
            acc_sp.at[pl.ds(row0, rpt)],
        )
        plsc.subcore_barrier()

        def start_gather(k, slot):
            sb = lax.rem(lax.div(k, IBLK), 2)
            o = lax.rem(k, IBLK) * MCHUNK
            pltpu.async_copy(
                y_hbm.at[rowi.at[sb, pl.ds(o, MCHUNK)]], bufs.at[slot],
                gsem.at[slot])

        def wait_gather(slot):
            pltpu.make_async_copy(
                y_hbm.at[rowi.at[0, pl.ds(0, MCHUNK)]], bufs.at[slot],
                gsem.at[slot]).wait()

        def start_scatter(k, slot):
            sb = lax.rem(lax.div(k, IBLK), 2)
            j = lax.rem(k, IBLK)
            pltpu.async_copy(
                bufs.at[slot], acc_sp.at[coli.at[sb, j, 0]], ssem.at[slot],
                add=True)

        def wait_scatter(slot):
            pltpu.make_async_copy(
                bufs.at[slot], acc_sp.at[coli.at[0, 0, 0]],
                ssem.at[slot]).wait()

        def scale(k, slot):
            sb = lax.rem(lax.div(k, IBLK), 2)
            o = lax.rem(k, IBLK) * MCHUNK
            sbv = jnp.full((L,), sb, dtype=jnp.int32)

            @pl.loop(0, MCHUNK, step=4)
            def _(e):
                for u in range(4):
                    idx = jnp.full((L,), o + e + u, dtype=jnp.int32)
                    wsp = plsc.load_gather(wv, [sbv, idx])
                    for f in range(HALF // L):
                        sl = pl.ds(f * L, L)
                        bufs[slot, e + u, sl] = bufs[slot, e + u, sl] * wsp

        load_block(0)
        start_gather(0, 0)
        start_gather(1, 1)

        @pl.loop(0, cpt)
        def _(k):
            @pl.when(jnp.logical_and(lax.rem(k, IBLK) == IBLK - 2,
                                     lax.div(k, IBLK) + 1 < nblk))
            def _():
                load_block(lax.div(k, IBLK) + 1)

            slot = lax.rem(k, 3)
            wait_gather(slot)
            scale(k, slot)
            start_scatter(k, slot)
            slot_r = lax.rem(k + 2, 3)

            @pl.when(k >= 1)
            def _():
                wait_scatter(slot_r)

            @pl.when(k + 2 < cpt)
            def _():
                start_gather(k + 2, slot_r)

        wait_scatter(lax.rem(cpt - 1, 3))

        plsc.subcore_barrier()
        pltpu.sync_copy(
            acc_sp.at[pl.ds(row0, rpt)],
            out_hbm.at[pl.ds(slab0 + row0, rpt)],
        )

    return msg_kernel(y_flat, row_flat, col3d, w_flat)


def _dis_from_degp(degp_blk):
    deg = jnp.sum(degp_blk, axis=0) + 1.0
    return jnp.where(deg > 0, lax.rsqrt(jnp.maximum(deg, 1e-12)), 0.0)


def _tc_y(x_pad, W, degp2):
    npad, d_in = x_pad.shape
    d_out = W.shape[0]
    br = npad // 8

    def body(x_ref, w_ref, degp_ref, y_ref):
        xl = lax.dot_general(
            x_ref[...], w_ref[...], (((1,), (1,)), ((), ())),
            preferred_element_type=jnp.float32,
        )
        y = xl * _dis_from_degp(degp_ref[...])[:, None]
        y_ref[...] = jnp.stack([y[:, :HALF], y[:, HALF:]], axis=0)

    return pl.pallas_call(
        body,
        grid=(npad // br,),
        in_specs=[
            pl.BlockSpec((br, d_in), lambda i: (i, 0)),
            pl.BlockSpec((d_out, d_in), lambda i: (0, 0)),
            pl.BlockSpec((NC * NS, br), lambda i: (0, i)),
        ],
        out_specs=pl.BlockSpec((NC, br, HALF), lambda i: (0, i, 0)),
        out_shape=jax.ShapeDtypeStruct((NC, npad, HALF), jnp.float32),
    )(x_pad, W, degp2)


def _tc_final(acc3, degp2, b2d):
    npad = acc3.shape[1]
    d_out = NC * HALF
    br = npad // 10

    def body(acc_ref, degp_ref, b_ref, o_ref):
        dis = _dis_from_degp(degp_ref[...])
        m = jnp.concatenate([acc_ref[0], acc_ref[1]], axis=1)
        o_ref[...] = m * dis[:, None] + b_ref[...]

    return pl.pallas_call(
        body,
        grid=(npad // br,),
        in_specs=[
            pl.BlockSpec((NC, br, HALF), lambda i: (0, i, 0)),
            pl.BlockSpec((NC * NS, br), lambda i: (0, i)),
            pl.BlockSpec((1, d_out), lambda i: (0, 0)),
        ],
        out_specs=pl.BlockSpec((br, d_out), lambda i: (i, 0)),
        out_shape=jax.ShapeDtypeStruct((npad, d_out), jnp.float32),
    )(acc3, degp2, b2d)


def kernel(x, edge_index, edge_weight, W, b):
    n = x.shape[0]
    e = edge_weight.shape[0]
    npad = -(-n // (NS * L)) * (NS * L)
    row = edge_index[0].astype(jnp.int32)
    col = edge_index[1].astype(jnp.int32)
    w = edge_weight.astype(jnp.float32)

    egrain = NS * MCHUNK * IBLK
    epad = -(-e // egrain) * egrain
    pad = epad - e
    pad_idx = (jnp.arange(pad, dtype=jnp.int32) * 37) % n
    row_flat = jnp.concatenate([row, pad_idx])
    col_flat = jnp.concatenate([col, pad_idx])
    col3d = col_flat.reshape(epad // MCHUNK, 1, MCHUNK)
    w_flat = jnp.concatenate([w, jnp.zeros((pad,), jnp.float32)])
    x_pad = jnp.concatenate(
        [x, jnp.zeros((npad - n, x.shape[1]), x.dtype)], axis=0)

    degp = _sc_deg(col_flat, w_flat, npad)
    degp2 = degp.reshape(NC * NS, npad)
    y = _tc_y(x_pad, W, degp2)
    acc = _sc_msg(y.reshape(NC * npad, HALF), row_flat, col3d, w_flat, npad)
    out = _tc_final(acc.reshape(NC, npad, HALF), degp2,
                    b.reshape(1, NC * HALF))
    return out[:n]


# speedup vs baseline: 2.1704x; 1.0006x over previous
import functools

import jax
import jax.numpy as jnp
from jax import lax
from jax.experimental import pallas as pl
from jax.experimental.pallas import tpu as pltpu
from jax.experimental.pallas import tpu_sc as plsc

L = 16
NC = 2
NS = 16
HALF = 128

_MESH = plsc.VectorSubcoreMesh(core_axis_name="c", subcore_axis_name="s")
_SC_PARAMS = pltpu.CompilerParams(needs_layout_passes=False)


def _sc_deg(col_flat, w_flat, npad):
    epad = col_flat.shape[0]
    ept = epad // (NC * NS)

    @functools.partial(
        pl.kernel,
        out_type=jax.ShapeDtypeStruct((NC * NS * npad,), jnp.float32),
        mesh=_MESH,
        compiler_params=_SC_PARAMS,
        scratch_types=[
            pltpu.VMEM((npad,), jnp.float32),
            pltpu.VMEM((ept,), jnp.int32),
            pltpu.VMEM((ept,), jnp.float32),
        ],
    )
    def deg_kernel(col_hbm, w_hbm, degp_hbm, deg_v, coli, wv):
        cid = lax.axis_index("c")
        sid = lax.axis_index("s")
        wid = cid * NS + sid
        zero = jnp.zeros((L,), jnp.float32)

        pltpu.sync_copy(col_hbm.at[pl.ds(wid * ept, ept)], coli)
        pltpu.sync_copy(w_hbm.at[pl.ds(wid * ept, ept)], wv)

        @pl.loop(0, npad // L)
        def _(g):
            deg_v[pl.ds(g * L, L)] = zero

        @pl.loop(0, ept // L)
        def _(g):
            sl = pl.ds(g * L, L)
            plsc.addupdate_scatter(deg_v, [coli[sl]], wv[sl])

        pltpu.sync_copy(deg_v, degp_hbm.at[pl.ds(wid * npad, npad)])

    return deg_kernel(col_flat, w_flat)


MCHUNK = 64
IBLK = 40


def _sc_msg(y_flat, row_flat, col3d, w_flat, npad):
    nchunks = col3d.shape[0]
    cpt = nchunks // NS
    nblk = cpt // IBLK
    eblk = IBLK * MCHUNK
    rpt = npad // NS

    @functools.partial(
        pl.kernel,
        out_type=jax.ShapeDtypeStruct((NC * npad, HALF), jnp.float32),
        mesh=_MESH,
        compiler_params=_SC_PARAMS,
        scratch_types=[
            pltpu.VMEM_SHARED((npad, HALF), jnp.float32),
            pltpu.VMEM((2, eblk), jnp.int32),
            pltpu.VMEM((2, IBLK, 1, MCHUNK), jnp.int32),
            pltpu.VMEM((2, eblk), jnp.float32),
            pltpu.VMEM((3, MCHUNK, HALF), jnp.float32),
            pltpu.SemaphoreType.DMA((3,)),
            pltpu.SemaphoreType.DMA((3,)),
        ],
    )
    def msg_kernel(y_hbm, row_hbm, col_hbm, w_hbm, out_hbm,
                   acc_sp, rowi, coli, wv, bufs, gsem, ssem):
        cid = lax.axis_index("c")
        sid = lax.axis_index("s")
        row0 = sid * rpt
        slab0 = cid * npad
        base = sid * cpt
        offv = jnp.full((L,), slab0, dtype=jnp.int32)

        def load_block(b):
            sb = lax.rem(b, 2)
            pltpu.sync_copy(row_hbm.at[pl.ds((base + b * IBLK) * MCHUNK, eblk)],
                            rowi.at[sb])
            pltpu.sync_copy(col_hbm.at[pl.ds(base + b * IBLK, IBLK)],
                            coli.at[sb])
            pltpu.sync_copy(w_hbm.at[pl.ds((base + b * IBLK) * MCHUNK, eblk)],
                            wv.at[sb])

            @pl.loop(0, eblk // L)
            def _(g):
                sl = pl.ds(g * L, L)
                rowi[sb, sl] = rowi[sb, sl] + offv

        pltpu.sync_copy(
            y_hbm.at[pl.ds(slab0 + row0, rpt)],
            acc_sp.at[pl.ds(row0, rpt)],
        )
        plsc.subcore_barrier()

        def start_gather(k, slot):
            sb = lax.rem(lax.div(k, IBLK), 2)
            o = lax.rem(k, IBLK) * MCHUNK
            pltpu.async_copy(
                y_hbm.at[rowi.at[sb, pl.ds(o, MCHUNK)]], bufs.at[slot],
                gsem.at[slot])

        def wait_gather(slot):
            pltpu.make_async_copy(
                y_hbm.at[rowi.at[0, pl.ds(0, MCHUNK)]], bufs.at[slot],
                gsem.at[slot]).wait()

        def start_scatter(k, slot):
            sb = lax.rem(lax.div(k, IBLK), 2)
            j = lax.rem(k, IBLK)
            pltpu.async_copy(
                bufs.at[slot], acc_sp.at[coli.at[sb, j, 0]], ssem.at[slot],
                add=True)

        def wait_scatter(slot):
            pltpu.make_async_copy(
                bufs.at[slot], acc_sp.at[coli.at[0, 0, 0]],
                ssem.at[slot]).wait()

        def scale(k, slot):
            sb = lax.rem(lax.div(k, IBLK), 2)
            o = lax.rem(k, IBLK) * MCHUNK
            sbv = jnp.full((L,), sb, dtype=jnp.int32)

            @pl.loop(0, MCHUNK, step=4)
            def _(e):
                for u in range(4):
                    idx = jnp.full((L,), o + e + u, dtype=jnp.int32)
                    wsp = plsc.load_gather(wv, [sbv, idx])
                    for f in range(HALF // L):
                        sl = pl.ds(f * L, L)
                        bufs[slot, e + u, sl] = bufs[slot, e + u, sl] * wsp

        load_block(0)
        start_gather(0, 0)
        start_gather(1, 1)

        @pl.loop(0, cpt)
        def _(k):
            @pl.when(jnp.logical_and(lax.rem(k, IBLK) == IBLK - 2,
                                     lax.div(k, IBLK) + 1 < nblk))
            def _():
                load_block(lax.div(k, IBLK) + 1)

            slot = lax.rem(k, 3)
            wait_gather(slot)
            scale(k, slot)
            start_scatter(k, slot)
            slot_r = lax.rem(k + 2, 3)

            @pl.when(k >= 1)
            def _():
                wait_scatter(slot_r)

            @pl.when(k + 2 < cpt)
            def _():
                start_gather(k + 2, slot_r)

        wait_scatter(lax.rem(cpt - 1, 3))

        plsc.subcore_barrier()
        pltpu.sync_copy(
            acc_sp.at[pl.ds(row0, rpt)],
            out_hbm.at[pl.ds(slab0 + row0, rpt)],
        )

    return msg_kernel(y_flat, row_flat, col3d, w_flat)


def _dis_from_degp(degp_blk):
    deg = jnp.sum(degp_blk, axis=0) + 1.0
    return jnp.where(deg > 0, lax.rsqrt(jnp.maximum(deg, 1e-12)), 0.0)


def _tc_y(x_pad, W, degp2):
    npad, d_in = x_pad.shape
    d_out = W.shape[0]
    br = npad // 8

    def body(x_ref, w_ref, degp_ref, y_ref):
        xl = lax.dot_general(
            x_ref[...], w_ref[...], (((1,), (1,)), ((), ())),
            preferred_element_type=jnp.float32,
        )
        y = xl * _dis_from_degp(degp_ref[...])[:, None]
        y_ref[...] = jnp.stack([y[:, :HALF], y[:, HALF:]], axis=0)

    return pl.pallas_call(
        body,
        grid=(npad // br,),
        in_specs=[
            pl.BlockSpec((br, d_in), lambda i: (i, 0)),
            pl.BlockSpec((d_out, d_in), lambda i: (0, 0)),
            pl.BlockSpec((NC * NS, br), lambda i: (0, i)),
        ],
        out_specs=pl.BlockSpec((NC, br, HALF), lambda i: (0, i, 0)),
        out_shape=jax.ShapeDtypeStruct((NC, npad, HALF), jnp.float32),
    )(x_pad, W, degp2)


def _tc_final(acc3, degp2, b2d):
    npad = acc3.shape[1]
    d_out = NC * HALF
    br = npad // 10

    def body(acc_ref, degp_ref, b_ref, o_ref):
        dis = _dis_from_degp(degp_ref[...])
        m = jnp.concatenate([acc_ref[0], acc_ref[1]], axis=1)
        o_ref[...] = m * dis[:, None] + b_ref[...]

    return pl.pallas_call(
        body,
        grid=(npad // br,),
        in_specs=[
            pl.BlockSpec((NC, br, HALF), lambda i: (0, i, 0)),
            pl.BlockSpec((NC * NS, br), lambda i: (0, i)),
            pl.BlockSpec((1, d_out), lambda i: (0, 0)),
        ],
        out_specs=pl.BlockSpec((br, d_out), lambda i: (i, 0)),
        out_shape=jax.ShapeDtypeStruct((npad, d_out), jnp.float32),
    )(acc3, degp2, b2d)


def kernel(x, edge_index, edge_weight, W, b):
    n = x.shape[0]
    e = edge_weight.shape[0]
    npad = -(-n // (NS * L)) * (NS * L)
    row = edge_index[0].astype(jnp.int32)
    col = edge_index[1].astype(jnp.int32)
    w = edge_weight.astype(jnp.float32)

    egrain = NS * MCHUNK * IBLK
    epad = -(-e // egrain) * egrain
    pad = epad - e
    pad_idx = (jnp.arange(pad, dtype=jnp.int32) * 37) % n
    row_flat = jnp.concatenate([row, pad_idx])
    col_flat = jnp.concatenate([col, pad_idx])
    col3d = col_flat.reshape(epad // MCHUNK, 1, MCHUNK)
    w_flat = jnp.concatenate([w, jnp.zeros((pad,), jnp.float32)])
    x_pad = jnp.concatenate(
        [x, jnp.zeros((npad - n, x.shape[1]), x.dtype)], axis=0)

    degp = _sc_deg(col_flat, w_flat, npad)
    degp2 = degp.reshape(NC * NS, npad)
    y = _tc_y(x_pad, W, degp2)
    acc = _sc_msg(y.reshape(NC * npad, HALF), row_flat, col3d, w_flat, npad)
    out = _tc_final(acc.reshape(NC, npad, HALF), degp2,
                    b.reshape(1, NC * HALF))
    return out[:n]
